# baseline (speedup 1.0000x reference)
import jax
import jax.numpy as jnp
from jax.experimental import pallas as pl
from jax.experimental.pallas import tpu as pltpu
from jax.experimental.pallas import tpu_sc as plsc

_W = 128
_WN = 8


def _sc_gather(idx, pe, R, D):
    mesh = plsc.VectorSubcoreMesh(core_axis_name="c", subcore_axis_name="s")

    @pl.kernel(out_type=jax.ShapeDtypeStruct((R, D), jnp.float32), mesh=mesh)
    def gather_k(i_hbm, pe_hbm, g_hbm):
        def body(i_vmem, g_vmem):
            pltpu.sync_copy(pe_hbm.at[i_vmem.at[0]], g_vmem)

        pltpu.emit_pipeline(
            body,
            grid=(R // _W,),
            in_specs=[pl.BlockSpec((1, _W), lambda i: (0, i))],
            out_specs=[pl.BlockSpec((_W, D), lambda i: (i, 0))],
            core_axis_name=("c", "s"),
            dimension_semantics=(pltpu.PARALLEL,),
        )(i_hbm, g_hbm)

    return gather_k(idx, pe)


def _tc_add(x3, g, B, N, L, D):
    def add_k(x_ref, g_ref, o_ref):
        for n in range(_WN):
            o_ref[n] = x_ref[n] + g_ref[pl.ds(n * L, L), :]

    return pl.pallas_call(
        add_k,
        grid=(B * N // _WN,),
        in_specs=[
            pl.BlockSpec((_WN, L, D), lambda j: (j, 0, 0)),
            pl.BlockSpec((_WN * L, D), lambda j: (j, 0)),
        ],
        out_specs=pl.BlockSpec((_WN, L, D), lambda j: (j, 0, 0)),
        out_shape=jax.ShapeDtypeStruct((B * N, L, D), jnp.float32),
    )(x3, g)


def kernel(input_emb, position, pe):
    B, N, L, D = input_emb.shape
    R = B * N * L
    x3 = input_emb.reshape(B * N, L, D)
    idx = position.reshape(1, R).astype(jnp.int32)

    @jax.jit
    def run(x3, idx, pe):
        g = _sc_gather(idx, pe, R, D)
        return _tc_add(x3, g, B, N, L, D)

    return run(x3, idx, pe).reshape(B, N, L, D)


# speedup vs baseline: 3.4174x; 3.4174x over previous
import jax
import jax.numpy as jnp
from jax.experimental import pallas as pl
from jax.experimental.pallas import tpu as pltpu
from jax.experimental.pallas import tpu_sc as plsc

_W = 128
_LANES = 16


def kernel(input_emb, position, pe):
    B, N, L, D = input_emb.shape
    R = B * N * L

    @jax.jit
    def run(input_emb, position, pe):
        x = input_emb.transpose(1, 2, 0, 3).reshape(R, D)
        idx = position.transpose(1, 2, 0).reshape(1, R).astype(jnp.int32)

        mesh = plsc.VectorSubcoreMesh(core_axis_name="c", subcore_axis_name="s")

        @pl.kernel(
            out_type=jax.ShapeDtypeStruct((R, D), jnp.float32),
            mesh=mesh,
            scratch_types=[pltpu.VMEM((_W, D), jnp.float32)],
        )
        def emb_add(x_hbm, i_hbm, pe_hbm, o_hbm, pe_rows):
            def body(i_vmem, x_vmem, o_vmem):
                pltpu.sync_copy(pe_hbm.at[i_vmem.at[0]], pe_rows)

                @pl.loop(0, _W)
                def _(r):
                    for c in range(0, D, _LANES):
                        o_vmem.at[r, pl.ds(c, _LANES)][...] = (
                            pe_rows.at[r, pl.ds(c, _LANES)][...]
                            + x_vmem.at[r, pl.ds(c, _LANES)][...]
                        )

            pltpu.emit_pipeline(
                body,
                grid=(R // _W,),
                in_specs=[
                    pl.BlockSpec((1, _W), lambda i: (0, i)),
                    pl.BlockSpec((_W, D), lambda i: (i, 0)),
                ],
                out_specs=[pl.BlockSpec((_W, D), lambda i: (i, 0))],
                core_axis_name=("c", "s"),
                dimension_semantics=(pltpu.PARALLEL,),
            )(i_hbm, x_hbm, o_hbm)

        out = emb_add(x, idx, pe)
        return out.reshape(N, L, B, D).transpose(2, 0, 1, 3)

    return run(input_emb, position, pe)


# speedup vs baseline: 4.7049x; 1.3767x over previous
import jax
import jax.numpy as jnp
from jax.experimental import pallas as pl
from jax.experimental.pallas import tpu as pltpu
from jax.experimental.pallas import tpu_sc as plsc

_W = 128
_TR = 1200


def _sc_gather(idx, pe, R, D):
    mesh = plsc.VectorSubcoreMesh(core_axis_name="c", subcore_axis_name="s")

    @pl.kernel(out_type=jax.ShapeDtypeStruct((R, D), jnp.float32), mesh=mesh)
    def gather_k(i_hbm, pe_hbm, g_hbm):
        def body(i_vmem, g_vmem):
            pltpu.sync_copy(pe_hbm.at[i_vmem.at[0]], g_vmem)

        pltpu.emit_pipeline(
            body,
            grid=(R // _W,),
            in_specs=[pl.BlockSpec((1, _W), lambda i: (0, i))],
            out_specs=[pl.BlockSpec((_W, D), lambda i: (i, 0))],
            core_axis_name=("c", "s"),
            dimension_semantics=(pltpu.PARALLEL,),
        )(i_hbm, g_hbm)

    return gather_k(idx, pe)


def _tc_add(x, g, R, D):
    def add_k(x_ref, g_ref, o_ref):
        o_ref[...] = x_ref[...] + g_ref[...]

    return pl.pallas_call(
        add_k,
        grid=(R // _TR,),
        in_specs=[
            pl.BlockSpec((_TR, D), lambda i: (i, 0)),
            pl.BlockSpec((_TR, D), lambda i: (i, 0)),
        ],
        out_specs=pl.BlockSpec((_TR, D), lambda i: (i, 0)),
        out_shape=jax.ShapeDtypeStruct((R, D), jnp.float32),
    )(x, g)


def kernel(input_emb, position, pe):
    B, N, L, D = input_emb.shape
    R = B * N * L

    @jax.jit
    def run(input_emb, position, pe):
        x = input_emb.transpose(1, 2, 0, 3).reshape(R, D)
        idx = position.transpose(1, 2, 0).reshape(1, R).astype(jnp.int32)
        g = _sc_gather(idx, pe, R, D)
        out = _tc_add(x, g, R, D)
        return out.reshape(N, L, B, D).transpose(2, 0, 1, 3)

    return run(input_emb, position, pe)


# speedup vs baseline: 5.5700x; 1.1839x over previous
import jax
import jax.numpy as jnp
from jax.experimental import pallas as pl
from jax.experimental.pallas import tpu as pltpu
from jax.experimental.pallas import tpu_sc as plsc

_W = 128
_TR = 2400


def _sc_gather(idx, pe, R, D):
    mesh = plsc.VectorSubcoreMesh(core_axis_name="c", subcore_axis_name="s")

    @pl.kernel(out_type=jax.ShapeDtypeStruct((R, D), jnp.float32), mesh=mesh)
    def gather_k(i_hbm, pe_hbm, g_hbm):
        def body(i_vmem, g_vmem):
            pltpu.sync_copy(pe_hbm.at[i_vmem.at[0]], g_vmem)

        pltpu.emit_pipeline(
            body,
            grid=(R // _W,),
            in_specs=[pl.BlockSpec((1, _W), lambda i: (0, i))],
            out_specs=[pl.BlockSpec((_W, D), lambda i: (i, 0))],
            core_axis_name=("c", "s"),
            dimension_semantics=(pltpu.PARALLEL,),
        )(i_hbm, g_hbm)

    return gather_k(idx, pe)


def _tc_add(x, g, R, D):
    def add_k(x_ref, g_ref, o_ref):
        o_ref[...] = x_ref[...] + g_ref[...]

    return pl.pallas_call(
        add_k,
        grid=(R // _TR,),
        in_specs=[
            pl.BlockSpec((_TR, D), lambda i: (i, 0)),
            pl.BlockSpec((_TR, D), lambda i: (i, 0)),
        ],
        out_specs=pl.BlockSpec((_TR, D), lambda i: (i, 0)),
        out_shape=jax.ShapeDtypeStruct((R, D), jnp.float32),
    )(x, g)


def kernel(input_emb, position, pe):
    B, N, L, D = input_emb.shape
    R = B * N * L

    @jax.jit
    def run(input_emb, position, pe):
        x = input_emb.transpose(1, 2, 0, 3).reshape(R, D)
        idx = position.transpose(1, 2, 0).reshape(1, R).astype(jnp.int32)
        g = _sc_gather(idx, pe, R, D)
        out = _tc_add(x, g, R, D)
        return out.reshape(N, L, B, D).transpose(2, 0, 1, 3)

    return run(input_emb, position, pe)


# speedup vs baseline: 5.8339x; 1.0474x over previous
import jax
import jax.numpy as jnp
from jax.experimental import pallas as pl
from jax.experimental.pallas import tpu as pltpu
from jax.experimental.pallas import tpu_sc as plsc

_W = 128
_TR = 4800


def _sc_gather(idx, pe, R, D):
    mesh = plsc.VectorSubcoreMesh(core_axis_name="c", subcore_axis_name="s")

    @pl.kernel(out_type=jax.ShapeDtypeStruct((R, D), jnp.float32), mesh=mesh)
    def gather_k(i_hbm, pe_hbm, g_hbm):
        def body(i_vmem, g_vmem):
            pltpu.sync_copy(pe_hbm.at[i_vmem.at[0]], g_vmem)

        pltpu.emit_pipeline(
            body,
            grid=(R // _W,),
            in_specs=[pl.BlockSpec((1, _W), lambda i: (0, i))],
            out_specs=[pl.BlockSpec((_W, D), lambda i: (i, 0))],
            core_axis_name=("c", "s"),
            dimension_semantics=(pltpu.PARALLEL,),
        )(i_hbm, g_hbm)

    return gather_k(idx, pe)


def _tc_add(x, g, R, D):
    def add_k(x_ref, g_ref, o_ref):
        o_ref[...] = x_ref[...] + g_ref[...]

    return pl.pallas_call(
        add_k,
        grid=(R // _TR,),
        in_specs=[
            pl.BlockSpec((_TR, D), lambda i: (i, 0)),
            pl.BlockSpec((_TR, D), lambda i: (i, 0)),
        ],
        out_specs=pl.BlockSpec((_TR, D), lambda i: (i, 0)),
        out_shape=jax.ShapeDtypeStruct((R, D), jnp.float32),
    )(x, g)


def kernel(input_emb, position, pe):
    B, N, L, D = input_emb.shape
    R = B * N * L

    @jax.jit
    def run(input_emb, position, pe):
        x = input_emb.transpose(1, 2, 0, 3).reshape(R, D)
        idx = position.transpose(1, 2, 0).reshape(1, R).astype(jnp.int32)
        g = _sc_gather(idx, pe, R, D)
        out = _tc_add(x, g, R, D)
        return out.reshape(N, L, B, D).transpose(2, 0, 1, 3)

    return run(input_emb, position, pe)


# speedup vs baseline: 5.8941x; 1.0103x over previous
import jax
import jax.numpy as jnp
from jax.experimental import pallas as pl
from jax.experimental.pallas import tpu as pltpu
from jax.experimental.pallas import tpu_sc as plsc

_W = 128
_TR = 7800


def _sc_gather(idx, pe, R, D):
    mesh = plsc.VectorSubcoreMesh(core_axis_name="c", subcore_axis_name="s")

    @pl.kernel(out_type=jax.ShapeDtypeStruct((R, D), jnp.float32), mesh=mesh)
    def gather_k(i_hbm, pe_hbm, g_hbm):
        def body(i_vmem, g_vmem):
            pltpu.sync_copy(pe_hbm.at[i_vmem.at[0]], g_vmem)

        pltpu.emit_pipeline(
            body,
            grid=(R // _W,),
            in_specs=[pl.BlockSpec((1, _W), lambda i: (0, i))],
            out_specs=[pl.BlockSpec((_W, D), lambda i: (i, 0))],
            core_axis_name=("c", "s"),
            dimension_semantics=(pltpu.PARALLEL,),
        )(i_hbm, g_hbm)

    return gather_k(idx, pe)


def _tc_add(x, g, R, D):
    def add_k(x_ref, g_ref, o_ref):
        o_ref[...] = x_ref[...] + g_ref[...]

    return pl.pallas_call(
        add_k,
        grid=(R // _TR,),
        in_specs=[
            pl.BlockSpec((_TR, D), lambda i: (i, 0)),
            pl.BlockSpec((_TR, D), lambda i: (i, 0)),
        ],
        out_specs=pl.BlockSpec((_TR, D), lambda i: (i, 0)),
        out_shape=jax.ShapeDtypeStruct((R, D), jnp.float32),
    )(x, g)


def kernel(input_emb, position, pe):
    B, N, L, D = input_emb.shape
    R = B * N * L

    @jax.jit
    def run(input_emb, position, pe):
        x = input_emb.transpose(1, 2, 0, 3).reshape(R, D)
        idx = position.transpose(1, 2, 0).reshape(1, R).astype(jnp.int32)
        g = _sc_gather(idx, pe, R, D)
        out = _tc_add(x, g, R, D)
        return out.reshape(N, L, B, D).transpose(2, 0, 1, 3)

    return run(input_emb, position, pe)
